# baseline (speedup 1.0000x reference)
import jax
import jax.numpy as jnp
from jax.experimental import pallas as pl
from jax.experimental.pallas import tpu as pltpu

_BETA = 1.0 / 9.0
_NEG_POS_RATIO = 3
_ROWS = 2560
_LANES = 128
_NCHUNK = 8
_GRID = 16
_RBLK = _ROWS // _NCHUNK


def _count_ge(keys, cand):
    return jnp.sum((keys >= cand).astype(jnp.int32))


def _loss_kernel(conf_ref, lab_ref, pred_ref, gt_ref, out_ref,
                 lmask_ref, np_ref, ce_ref, sl_ref, base_ref, keff_ref):
    i = pl.program_id(0)

    @pl.when(i == 0)
    def _init():
        np_ref[0] = 0
        ce_ref[0] = 0.0
        sl_ref[0] = 0.0

    @pl.when(i < _NCHUNK)
    def _conf_phase():
        c0 = conf_ref[0::2, :]
        c1 = conf_ref[1::2, :]
        x = c1 - c0
        sp = jnp.maximum(x, 0.0) + jnp.log1p(jnp.exp(-jnp.abs(x)))
        pos = lab_ref[:] > 0
        lmask_ref[pl.ds(i * _RBLK, _RBLK), :] = jnp.where(pos, -1.0, sp)
        np_ref[0] += jnp.sum(pos.astype(jnp.int32))
        ce_ref[0] += jnp.sum(jnp.where(pos, sp - x, 0.0))

    @pl.when((i & 1) == 1)
    def _sl1_phase():
        d1 = jnp.abs(pred_ref[1::4, :] - gt_ref[1::4, :])
        d3 = jnp.abs(pred_ref[3::4, :] - gt_ref[3::4, :])
        sl1 = (jnp.where(d1 < _BETA, 0.5 / _BETA * d1 * d1,
                         d1 - 0.5 * _BETA)
               + jnp.where(d3 < _BETA, 0.5 / _BETA * d3 * d3,
                           d3 - 0.5 * _BETA))
        c = i // 2
        pos4 = lmask_ref[pl.ds(c * _RBLK, _RBLK), :] < 0.0
        sl_ref[0] += jnp.sum(jnp.where(pos4, sl1, 0.0))

    @pl.when(i == _NCHUNK - 1)
    def _keff():
        num_pos = np_ref[0]
        keff_ref[0] = jnp.minimum(num_pos * _NEG_POS_RATIO,
                                  _ROWS * _LANES - num_pos)
        base_ref[0] = 0

    @pl.when(i >= _NCHUNK)
    def _search_phase():
        k_eff = keff_ref[0]
        keys = jax.lax.bitcast_convert_type(lmask_ref[:], jnp.int32)

        def two_bit(base, j):
            b_lo = jax.lax.shift_left(jnp.int32(1), j)
            ca = base + b_lo
            cb = base + 2 * b_lo
            cc = base + 3 * b_lo
            na = _count_ge(keys, ca)
            nb = _count_ge(keys, cb)
            nc = _count_ge(keys, cc)
            return jnp.where(
                nc >= k_eff, cc,
                jnp.where(nb >= k_eff, cb,
                          jnp.where(na >= k_eff, ca, base)))

        @pl.when(i == _NCHUNK)
        def _first():
            top = jnp.int32(1 << 30)
            b0 = jnp.where(_count_ge(keys, top) >= k_eff, top, jnp.int32(0))
            base_ref[0] = two_bit(b0, jnp.int32(28))

        @pl.when(i > _NCHUNK)
        def _rest():
            ja = 26 - 4 * (i - (_NCHUNK + 1))
            base_ref[0] = two_bit(two_bit(base_ref[0], ja), ja - 2)

    @pl.when(i == _GRID - 1)
    def _finalize():
        num_pos = np_ref[0]
        k_eff = keff_ref[0]
        base = base_ref[0]
        keys = jax.lax.bitcast_convert_type(lmask_ref[:], jnp.int32)
        gt_m = keys > base
        count_gt = jnp.sum(gt_m.astype(jnp.int32))
        s_gt = jnp.sum(jnp.where(gt_m, lmask_ref[:], 0.0))
        l_thr = jax.lax.bitcast_convert_type(base, jnp.float32)
        remaining = (k_eff - count_gt).astype(jnp.float32)
        s_neg = jnp.where(k_eff == 0, 0.0, s_gt + remaining * l_thr)

        n_sel = (num_pos + k_eff).astype(jnp.float32)
        loss_cls = jnp.clip((ce_ref[0] + s_neg) / jnp.maximum(n_sel, 1.0),
                            0.0, 5.0)
        loss_ver = jnp.clip(
            sl_ref[0] / jnp.maximum(2.0 * num_pos.astype(jnp.float32), 1.0),
            0.0, 5.0)
        loss_total = loss_ver + loss_cls

        row = jax.lax.broadcasted_iota(jnp.int32, (8, 128), 0)
        col = jax.lax.broadcasted_iota(jnp.int32, (8, 128), 1)
        out_ref[:] = (jnp.where((row == 0) & (col == 0), loss_total, 0.0)
                      + jnp.where((row == 0) & (col == 1), loss_cls, 0.0)
                      + jnp.where((row == 0) & (col == 2), loss_ver, 0.0))


@jax.jit
def kernel(confidence, predicted_locations, labels, gt_locations):
    B, A = labels.shape
    nt = A // _LANES
    conf_v = confidence.reshape(B, nt, _LANES, 2).transpose(0, 1, 3, 2) \
        .reshape(2 * _ROWS, _LANES)
    pred_v = predicted_locations.reshape(B, nt, _LANES, 4) \
        .transpose(0, 1, 3, 2).reshape(4 * _ROWS, _LANES)
    gt_v = gt_locations.reshape(B, nt, _LANES, 4) \
        .transpose(0, 1, 3, 2).reshape(4 * _ROWS, _LANES)
    lab2 = labels.reshape(_ROWS, _LANES)

    c7 = _NCHUNK - 1
    out = pl.pallas_call(
        _loss_kernel,
        grid=(_GRID,),
        in_specs=[
            pl.BlockSpec((2 * _RBLK, _LANES),
                         lambda i: (jnp.minimum(i, c7), 0)),
            pl.BlockSpec((_RBLK, _LANES),
                         lambda i: (jnp.minimum(i, c7), 0)),
            pl.BlockSpec((4 * _RBLK, _LANES), lambda i: (i // 2, 0)),
            pl.BlockSpec((4 * _RBLK, _LANES), lambda i: (i // 2, 0)),
        ],
        out_specs=pl.BlockSpec((8, 128), lambda i: (0, 0)),
        out_shape=jax.ShapeDtypeStruct((8, 128), jnp.float32),
        scratch_shapes=[
            pltpu.VMEM((_ROWS, _LANES), jnp.float32),
            pltpu.SMEM((1,), jnp.int32),
            pltpu.SMEM((1,), jnp.float32),
            pltpu.SMEM((1,), jnp.float32),
            pltpu.SMEM((1,), jnp.int32),
            pltpu.SMEM((1,), jnp.int32),
        ],
    )(conf_v, lab2, pred_v, gt_v)

    loss_total = out[0, 0]
    loss_cls = out[0, 1]
    loss_ver = out[0, 2]
    loss_refine = jnp.zeros((), jnp.float32)
    return (loss_total, loss_cls, loss_ver, loss_refine)


# speedup vs baseline: 1.8125x; 1.8125x over previous
import jax
import jax.numpy as jnp
from jax.experimental import pallas as pl
from jax.experimental.pallas import tpu as pltpu

_BETA = 1.0 / 9.0
_NEG_POS_RATIO = 3
_ROWS = 2560
_LANES = 128
_GRID = 8
_RBLK = _ROWS // _GRID


def _loss_kernel(conf_ref, lab_ref, pred_ref, gt_ref, out_ref,
                 lmask_ref, np_ref, ce_ref, sl_ref, sn_ref):
    i = pl.program_id(0)

    @pl.when(i == 0)
    def _init():
        np_ref[0] = 0
        ce_ref[0] = 0.0
        sl_ref[0] = 0.0
        sn_ref[0] = 0.0

    c0 = conf_ref[0::2, :]
    c1 = conf_ref[1::2, :]
    x = c1 - c0
    sp = jnp.maximum(x, 0.0) + jnp.log1p(jnp.exp(-jnp.abs(x)))
    pos = lab_ref[:] > 0
    posf = pos.astype(jnp.float32)

    lmask_ref[pl.ds(i * _RBLK, _RBLK), :] = jnp.where(pos, -1.0, sp)

    np_ref[0] += jnp.sum(pos.astype(jnp.int32))
    ce_ref[0] += jnp.sum(jnp.where(pos, sp - x, 0.0))
    sn_ref[0] += jnp.sum(jnp.where(pos, 0.0, sp))

    d1 = jnp.abs(pred_ref[1::4, :] - gt_ref[1::4, :])
    d3 = jnp.abs(pred_ref[3::4, :] - gt_ref[3::4, :])
    sl1 = jnp.where(d1 < _BETA, 0.5 / _BETA * d1 * d1, d1 - 0.5 * _BETA) + \
          jnp.where(d3 < _BETA, 0.5 / _BETA * d3 * d3, d3 - 0.5 * _BETA)
    sl_ref[0] += jnp.sum(sl1 * posf)

    @pl.when(i == _GRID - 1)
    def _finalize():
        num_pos = np_ref[0]
        n_total = _ROWS * _LANES
        num_neg = n_total - num_pos
        k_eff = jnp.minimum(num_pos * _NEG_POS_RATIO, num_neg)

        def _emit(s_neg):
            n_sel = (num_pos + k_eff).astype(jnp.float32)
            loss_cls = jnp.clip(
                (ce_ref[0] + s_neg) / jnp.maximum(n_sel, 1.0), 0.0, 5.0)
            loss_ver = jnp.clip(
                sl_ref[0] / jnp.maximum(2.0 * num_pos.astype(jnp.float32),
                                        1.0),
                0.0, 5.0)
            loss_total = loss_ver + loss_cls
            row = jax.lax.broadcasted_iota(jnp.int32, (8, 128), 0)
            col = jax.lax.broadcasted_iota(jnp.int32, (8, 128), 1)
            out_ref[:] = (
                jnp.where((row == 0) & (col == 0), loss_total, 0.0)
                + jnp.where((row == 0) & (col == 1), loss_cls, 0.0)
                + jnp.where((row == 0) & (col == 2), loss_ver, 0.0))

        @pl.when(k_eff >= num_neg)
        def _all_selected():
            _emit(sn_ref[0])

        @pl.when(k_eff < num_neg)
        def _search():
            def search_body(it, base):
                j = 28 - 2 * it
                b_lo = jax.lax.shift_left(jnp.int32(1), j)
                ca = base + b_lo
                cb = base + 2 * b_lo
                cc = base + 3 * b_lo
                keys = jax.lax.bitcast_convert_type(lmask_ref[:], jnp.int32)
                na = jnp.sum((keys >= ca).astype(jnp.int32))
                nb = jnp.sum((keys >= cb).astype(jnp.int32))
                nc = jnp.sum((keys >= cc).astype(jnp.int32))
                return jnp.where(
                    nc >= k_eff, cc,
                    jnp.where(nb >= k_eff, cb,
                              jnp.where(na >= k_eff, ca, base)))

            keys0 = jax.lax.bitcast_convert_type(lmask_ref[:], jnp.int32)
            top = jnp.int32(1 << 30)
            n_top = jnp.sum((keys0 >= top).astype(jnp.int32))
            base0 = jnp.where(n_top >= k_eff, top, jnp.int32(0))
            base = jax.lax.fori_loop(0, 15, search_body, base0)

            keys = jax.lax.bitcast_convert_type(lmask_ref[:], jnp.int32)
            gt_m = keys > base
            count_gt = jnp.sum(gt_m.astype(jnp.int32))
            s_gt = jnp.sum(jnp.where(gt_m, lmask_ref[:], 0.0))
            l_thr = jax.lax.bitcast_convert_type(base, jnp.float32)
            remaining = (k_eff - count_gt).astype(jnp.float32)
            _emit(jnp.where(k_eff == 0, 0.0, s_gt + remaining * l_thr))


@jax.jit
def kernel(confidence, predicted_locations, labels, gt_locations):
    B, A = labels.shape
    nt = A // _LANES
    conf_v = confidence.reshape(B, nt, _LANES, 2).transpose(0, 1, 3, 2) \
        .reshape(2 * _ROWS, _LANES)
    pred_v = predicted_locations.reshape(B, nt, _LANES, 4) \
        .transpose(0, 1, 3, 2).reshape(4 * _ROWS, _LANES)
    gt_v = gt_locations.reshape(B, nt, _LANES, 4) \
        .transpose(0, 1, 3, 2).reshape(4 * _ROWS, _LANES)
    lab2 = labels.reshape(_ROWS, _LANES)

    out = pl.pallas_call(
        _loss_kernel,
        grid=(_GRID,),
        in_specs=[
            pl.BlockSpec((2 * _RBLK, _LANES), lambda i: (i, 0)),
            pl.BlockSpec((_RBLK, _LANES), lambda i: (i, 0)),
            pl.BlockSpec((4 * _RBLK, _LANES), lambda i: (i, 0)),
            pl.BlockSpec((4 * _RBLK, _LANES), lambda i: (i, 0)),
        ],
        out_specs=pl.BlockSpec((8, 128), lambda i: (0, 0)),
        out_shape=jax.ShapeDtypeStruct((8, 128), jnp.float32),
        scratch_shapes=[
            pltpu.VMEM((_ROWS, _LANES), jnp.float32),
            pltpu.SMEM((1,), jnp.int32),
            pltpu.SMEM((1,), jnp.float32),
            pltpu.SMEM((1,), jnp.float32),
            pltpu.SMEM((1,), jnp.float32),
        ],
    )(conf_v, lab2, pred_v, gt_v)

    loss_total = out[0, 0]
    loss_cls = out[0, 1]
    loss_ver = out[0, 2]
    loss_refine = jnp.zeros((), jnp.float32)
    return (loss_total, loss_cls, loss_ver, loss_refine)
